# baseline (speedup 1.0000x reference)
import functools

import jax
import jax.numpy as jnp
from jax import lax
from jax.experimental import pallas as pl
from jax.experimental.pallas import tpu as pltpu
from jax.experimental.pallas import tpu_sc as plsc

N_PAIRS = 65536
N_NODES = 100000
FEAT = 320
HIDDEN = 128
CHUNK = 128
N_SEG = 4
SEG = N_PAIRS // N_SEG


def _embed_body(dt_ref, pt_ref, wa_ref, wb_ref, od_ref, op_ref):
    dn = (((0,), (0,)), ((), ()))
    od_ref[...] = lax.dot_general(dt_ref[...], wa_ref[...], dn,
                                  preferred_element_type=jnp.float32)
    op_ref[...] = lax.dot_general(pt_ref[...], wb_ref[...], dn,
                                  preferred_element_type=jnp.float32)


def _tc_embed(d_t, p_t, wa, wb):
    blk = 6144
    out = jax.ShapeDtypeStruct((N_NODES, HIDDEN), jnp.float32)
    return pl.pallas_call(
        _embed_body,
        grid=((N_NODES + blk - 1) // blk,),
        in_specs=[
            pl.BlockSpec((FEAT, blk), lambda i: (0, i)),
            pl.BlockSpec((FEAT, blk), lambda i: (0, i)),
            pl.BlockSpec((FEAT, HIDDEN), lambda i: (0, 0)),
            pl.BlockSpec((FEAT, HIDDEN), lambda i: (0, 0)),
        ],
        out_specs=[
            pl.BlockSpec((blk, HIDDEN), lambda i: (i, 0)),
            pl.BlockSpec((blk, HIDDEN), lambda i: (i, 0)),
        ],
        out_shape=[out, out],
        compiler_params=pltpu.CompilerParams(
            dimension_semantics=("arbitrary",),
        ),
    )(d_t, p_t, wa, wb)


_NSLOT = 6


def _sc_gather_body(dp_hbm, pp_hbm, ds_hbm, out_hbm,
                    idx_v, rows_0, rows_1, rows_2, rows_3, rows_4, rows_5,
                    g0, g1, g2, g3, g4, g5, w0, w1, w2, w3, w4, w5, *, lo):
    nc = 2
    wid = lax.axis_index("s") * nc + lax.axis_index("c")
    per_w = SEG // 32
    n_chunks = per_w // CHUNK
    base = wid * per_w

    pltpu.sync_copy(ds_hbm.at[0, pl.ds(lo + base, per_w)],
                    idx_v.at[pl.ds(0, per_w)])
    pltpu.sync_copy(ds_hbm.at[1, pl.ds(lo + base, per_w)],
                    idx_v.at[pl.ds(per_w, per_w)])

    rows = (rows_0, rows_1, rows_2, rows_3, rows_4, rows_5)
    gsem = (g0, g1, g2, g3, g4, g5)
    wsem = (w0, w1, w2, w3, w4, w5)

    def start_d(c):
        s = c % _NSLOT
        pltpu.async_copy(dp_hbm.at[idx_v.at[pl.ds(c * CHUNK, CHUNK)]],
                         rows[s], gsem[s])

    def wait_g(c):
        s = c % _NSLOT
        pltpu.make_async_copy(dp_hbm.at[pl.ds(0, CHUNK)], rows[s],
                              gsem[s]).wait()

    def start_p(c):
        s = c % _NSLOT
        pltpu.async_copy(pp_hbm.at[idx_v.at[pl.ds(per_w + c * CHUNK, CHUNK)]],
                         rows[s], gsem[s], add=True)

    def start_wb(c):
        s = c % _NSLOT
        off = base + c * CHUNK
        pltpu.async_copy(rows[s], out_hbm.at[pl.ds(off, CHUNK)], wsem[s])

    def wait_wb(c):
        s = c % _NSLOT
        pltpu.make_async_copy(rows[s], out_hbm.at[pl.ds(0, CHUNK)],
                              wsem[s]).wait()

    wb_drained = set()
    for c in range(min(_NSLOT, n_chunks)):
        start_d(c)
    for c in range(n_chunks):
        wait_g(c)
        start_p(c)
        if c >= 1:
            wait_g(c - 1)
            start_wb(c - 1)
        if c >= 2 and c + _NSLOT - 2 < n_chunks:
            wait_wb(c - 2)
            wb_drained.add(c - 2)
            start_d(c + _NSLOT - 2)
    wait_g(n_chunks - 1)
    start_wb(n_chunks - 1)
    for c in range(n_chunks):
        if c not in wb_drained:
            wait_wb(c)


def _sc_gather(dp, pp, ds_t, lo):
    mesh = plsc.VectorSubcoreMesh(core_axis_name="c", subcore_axis_name="s")
    per_w = SEG // 32
    return pl.kernel(
        functools.partial(_sc_gather_body, lo=lo),
        out_type=jax.ShapeDtypeStruct((SEG, HIDDEN), jnp.float32),
        mesh=mesh,
        scratch_types=(
            [pltpu.VMEM((2 * per_w,), jnp.int32)]
            + [pltpu.VMEM((CHUNK, HIDDEN), jnp.float32)] * _NSLOT
            + [pltpu.SemaphoreType.DMA] * (2 * _NSLOT)
        ),
        name=f"sc_gather_seg{lo}",
    )(dp, pp, ds_t)


def _head_body(xs_ref, b1_ref, w2_ref, b2_ref, o_ref):
    h = jnp.tanh(xs_ref[...] + b1_ref[...])
    logits = lax.dot_general(
        w2_ref[...], h,
        dimension_numbers=(((0,), (1,)), ((), ())),
        preferred_element_type=jnp.float32)
    logits += b2_ref[...]
    m = jnp.max(logits, axis=0, keepdims=True)
    lse = m + jnp.log(jnp.sum(jnp.exp(logits - m), axis=0, keepdims=True))
    o_ref[...] = logits - lse


def _tc_head(xs, b1, w2, b2):
    blk = 4096
    return pl.pallas_call(
        _head_body,
        grid=(SEG // blk,),
        in_specs=[
            pl.BlockSpec((blk, HIDDEN), lambda i: (i, 0)),
            pl.BlockSpec((1, HIDDEN), lambda i: (0, 0)),
            pl.BlockSpec((HIDDEN, 2), lambda i: (0, 0)),
            pl.BlockSpec((2, 1), lambda i: (0, 0)),
        ],
        out_specs=pl.BlockSpec((2, blk), lambda i: (0, i)),
        out_shape=jax.ShapeDtypeStruct((2, SEG), jnp.float32),
        compiler_params=pltpu.CompilerParams(
            dimension_semantics=("arbitrary",),
        ),
    )(xs, b1, w2, b2)


def kernel(graph, dataset_index, iftrain, d, p, W1, b1, W2, b2):
    ds_t = dataset_index.astype(jnp.int32).T
    dp, pp = _tc_embed(d.T, p.T, W1[:FEAT], W1[FEAT:])
    b1r = b1.reshape(1, HIDDEN)
    b2r = b2.reshape(2, 1)
    outs = []
    for s in range(N_SEG):
        xs = _sc_gather(dp, pp, ds_t, s * SEG)
        outs.append(_tc_head(xs, b1r, W2, b2r))
    return jnp.concatenate(outs, axis=1).T


# speedup vs baseline: 1.0648x; 1.0648x over previous
import functools

import jax
import jax.numpy as jnp
from jax import lax
from jax.experimental import pallas as pl
from jax.experimental.pallas import tpu as pltpu
from jax.experimental.pallas import tpu_sc as plsc

N_PAIRS = 65536
N_NODES = 100000
FEAT = 320
HIDDEN = 128
CHUNK = 128
N_SEG = 2
SEG = N_PAIRS // N_SEG


def _embed_body(dt_ref, pt_ref, wa_ref, wb_ref, od_ref, op_ref):
    dn = (((0,), (0,)), ((), ()))
    od_ref[...] = lax.dot_general(dt_ref[...], wa_ref[...], dn,
                                  preferred_element_type=jnp.float32)
    op_ref[...] = lax.dot_general(pt_ref[...], wb_ref[...], dn,
                                  preferred_element_type=jnp.float32)


def _tc_embed(d_t, p_t, wa, wb):
    blk = 6144
    out = jax.ShapeDtypeStruct((N_NODES, HIDDEN), jnp.float32)
    return pl.pallas_call(
        _embed_body,
        grid=((N_NODES + blk - 1) // blk,),
        in_specs=[
            pl.BlockSpec((FEAT, blk), lambda i: (0, i)),
            pl.BlockSpec((FEAT, blk), lambda i: (0, i)),
            pl.BlockSpec((FEAT, HIDDEN), lambda i: (0, 0)),
            pl.BlockSpec((FEAT, HIDDEN), lambda i: (0, 0)),
        ],
        out_specs=[
            pl.BlockSpec((blk, HIDDEN), lambda i: (i, 0)),
            pl.BlockSpec((blk, HIDDEN), lambda i: (i, 0)),
        ],
        out_shape=[out, out],
        compiler_params=pltpu.CompilerParams(
            dimension_semantics=("arbitrary",),
        ),
    )(d_t, p_t, wa, wb)


_NSLOT = 7


def _sc_gather_body(dp_hbm, pp_hbm, ds_hbm, out_hbm,
                    idx_v, rows_0, rows_1, rows_2, rows_3, rows_4, rows_5,
                    rows_6, g0, g1, g2, g3, g4, g5, g6,
                    w0, w1, w2, w3, w4, w5, w6, *, lo):
    nc = 2
    wid = lax.axis_index("s") * nc + lax.axis_index("c")
    per_w = SEG // 32
    n_chunks = per_w // CHUNK
    base = wid * per_w

    pltpu.sync_copy(ds_hbm.at[0, pl.ds(lo + base, per_w)],
                    idx_v.at[pl.ds(0, per_w)])
    pltpu.sync_copy(ds_hbm.at[1, pl.ds(lo + base, per_w)],
                    idx_v.at[pl.ds(per_w, per_w)])

    rows = (rows_0, rows_1, rows_2, rows_3, rows_4, rows_5, rows_6)
    gsem = (g0, g1, g2, g3, g4, g5, g6)
    wsem = (w0, w1, w2, w3, w4, w5, w6)

    def start_d(c):
        s = c % _NSLOT
        pltpu.async_copy(dp_hbm.at[idx_v.at[pl.ds(c * CHUNK, CHUNK)]],
                         rows[s], gsem[s])

    def wait_g(c):
        s = c % _NSLOT
        pltpu.make_async_copy(dp_hbm.at[pl.ds(0, CHUNK)], rows[s],
                              gsem[s]).wait()

    def start_p(c):
        s = c % _NSLOT
        pltpu.async_copy(pp_hbm.at[idx_v.at[pl.ds(per_w + c * CHUNK, CHUNK)]],
                         rows[s], gsem[s], add=True)

    def start_wb(c):
        s = c % _NSLOT
        off = base + c * CHUNK
        pltpu.async_copy(rows[s], out_hbm.at[pl.ds(off, CHUNK)], wsem[s])

    def wait_wb(c):
        s = c % _NSLOT
        pltpu.make_async_copy(rows[s], out_hbm.at[pl.ds(0, CHUNK)],
                              wsem[s]).wait()

    wb_drained = set()
    for c in range(min(_NSLOT, n_chunks)):
        start_d(c)
    for c in range(n_chunks):
        wait_g(c)
        start_p(c)
        if c >= 1:
            wait_g(c - 1)
            start_wb(c - 1)
        if c >= 2 and c + _NSLOT - 2 < n_chunks:
            wait_wb(c - 2)
            wb_drained.add(c - 2)
            start_d(c + _NSLOT - 2)
    wait_g(n_chunks - 1)
    start_wb(n_chunks - 1)
    for c in range(n_chunks):
        if c not in wb_drained:
            wait_wb(c)


def _sc_gather(dp, pp, ds_t, lo):
    mesh = plsc.VectorSubcoreMesh(core_axis_name="c", subcore_axis_name="s")
    per_w = SEG // 32
    return pl.kernel(
        functools.partial(_sc_gather_body, lo=lo),
        out_type=jax.ShapeDtypeStruct((SEG, HIDDEN), jnp.float32),
        mesh=mesh,
        scratch_types=(
            [pltpu.VMEM((2 * per_w,), jnp.int32)]
            + [pltpu.VMEM((CHUNK, HIDDEN), jnp.float32)] * _NSLOT
            + [pltpu.SemaphoreType.DMA] * (2 * _NSLOT)
        ),
        name=f"sc_gather_seg{lo}",
    )(dp, pp, ds_t)


def _head_body(xs_ref, b1_ref, w2_ref, b2_ref, o_ref):
    h = jnp.tanh(xs_ref[...] + b1_ref[...])
    logits = lax.dot_general(
        w2_ref[...], h,
        dimension_numbers=(((0,), (1,)), ((), ())),
        preferred_element_type=jnp.float32)
    logits += b2_ref[...]
    m = jnp.max(logits, axis=0, keepdims=True)
    lse = m + jnp.log(jnp.sum(jnp.exp(logits - m), axis=0, keepdims=True))
    o_ref[...] = logits - lse


def _tc_head(xs, b1, w2, b2):
    blk = 8192
    return pl.pallas_call(
        _head_body,
        grid=(SEG // blk,),
        in_specs=[
            pl.BlockSpec((blk, HIDDEN), lambda i: (i, 0)),
            pl.BlockSpec((1, HIDDEN), lambda i: (0, 0)),
            pl.BlockSpec((HIDDEN, 2), lambda i: (0, 0)),
            pl.BlockSpec((2, 1), lambda i: (0, 0)),
        ],
        out_specs=pl.BlockSpec((2, blk), lambda i: (0, i)),
        out_shape=jax.ShapeDtypeStruct((2, SEG), jnp.float32),
        compiler_params=pltpu.CompilerParams(
            dimension_semantics=("arbitrary",),
        ),
    )(xs, b1, w2, b2)


def kernel(graph, dataset_index, iftrain, d, p, W1, b1, W2, b2):
    ds_t = dataset_index.astype(jnp.int32).T
    dp, pp = _tc_embed(d.T, p.T, W1[:FEAT], W1[FEAT:])
    b1r = b1.reshape(1, HIDDEN)
    b2r = b2.reshape(2, 1)
    outs = []
    for s in range(N_SEG):
        xs = _sc_gather(dp, pp, ds_t, s * SEG)
        outs.append(_tc_head(xs, b1r, W2, b2r))
    return jnp.concatenate(outs, axis=1).T


# speedup vs baseline: 1.0681x; 1.0032x over previous
import functools

import jax
import jax.numpy as jnp
from jax import lax
from jax.experimental import pallas as pl
from jax.experimental.pallas import tpu as pltpu
from jax.experimental.pallas import tpu_sc as plsc

N_PAIRS = 65536
N_NODES = 100000
FEAT = 320
HIDDEN = 128
CHUNK = 128
N_SEG = 2
SEG = N_PAIRS // N_SEG


def _embed_body(dt_ref, pt_ref, wa_ref, wb_ref, od_ref, op_ref):
    dn = (((0,), (0,)), ((), ()))
    od_ref[...] = lax.dot_general(dt_ref[...], wa_ref[...], dn,
                                  preferred_element_type=jnp.float32)
    op_ref[...] = lax.dot_general(pt_ref[...], wb_ref[...], dn,
                                  preferred_element_type=jnp.float32)


def _tc_embed(d_t, p_t, wa, wb):
    blk = 6144
    out = jax.ShapeDtypeStruct((N_NODES, HIDDEN), jnp.float32)
    return pl.pallas_call(
        _embed_body,
        grid=((N_NODES + blk - 1) // blk,),
        in_specs=[
            pl.BlockSpec((FEAT, blk), lambda i: (0, i)),
            pl.BlockSpec((FEAT, blk), lambda i: (0, i)),
            pl.BlockSpec((FEAT, HIDDEN), lambda i: (0, 0)),
            pl.BlockSpec((FEAT, HIDDEN), lambda i: (0, 0)),
        ],
        out_specs=[
            pl.BlockSpec((blk, HIDDEN), lambda i: (i, 0)),
            pl.BlockSpec((blk, HIDDEN), lambda i: (i, 0)),
        ],
        out_shape=[out, out],
        compiler_params=pltpu.CompilerParams(
            dimension_semantics=("arbitrary",),
        ),
    )(d_t, p_t, wa, wb)


_NSLOT = 7


def _sc_gather_body(dp_hbm, pp_hbm, ds_hbm, out_hbm,
                    idx_v, rows_0, rows_1, rows_2, rows_3, rows_4, rows_5,
                    rows_6, g0, g1, g2, g3, g4, g5, g6,
                    w0, w1, w2, w3, w4, w5, w6, *, lo):
    nc = 2
    wid = lax.axis_index("s") * nc + lax.axis_index("c")
    per_w = SEG // 32
    n_chunks = per_w // CHUNK
    base = wid * per_w

    pltpu.sync_copy(ds_hbm.at[0, pl.ds(lo + base, per_w)],
                    idx_v.at[pl.ds(0, per_w)])
    pltpu.sync_copy(ds_hbm.at[1, pl.ds(lo + base, per_w)],
                    idx_v.at[pl.ds(per_w, per_w)])

    rows = (rows_0, rows_1, rows_2, rows_3, rows_4, rows_5, rows_6)
    gsem = (g0, g1, g2, g3, g4, g5, g6)
    wsem = (w0, w1, w2, w3, w4, w5, w6)

    def start_d(c):
        s = c % _NSLOT
        pltpu.async_copy(dp_hbm.at[idx_v.at[pl.ds(c * CHUNK, CHUNK)]],
                         rows[s], gsem[s])

    def wait_g(c):
        s = c % _NSLOT
        pltpu.make_async_copy(dp_hbm.at[pl.ds(0, CHUNK)], rows[s],
                              gsem[s]).wait()

    def start_p(c):
        s = c % _NSLOT
        pltpu.async_copy(pp_hbm.at[idx_v.at[pl.ds(per_w + c * CHUNK, CHUNK)]],
                         rows[s], gsem[s], add=True)

    def start_wb(c):
        s = c % _NSLOT
        off = base + c * CHUNK
        pltpu.async_copy(rows[s], out_hbm.at[pl.ds(off, CHUNK)], wsem[s])

    def wait_wb(c):
        s = c % _NSLOT
        pltpu.make_async_copy(rows[s], out_hbm.at[pl.ds(0, CHUNK)],
                              wsem[s]).wait()

    wb_drained = set()
    for c in range(min(_NSLOT, n_chunks)):
        start_d(c)
    for c in range(n_chunks):
        wait_g(c)
        start_p(c)
        if c >= 1:
            wait_g(c - 1)
            start_wb(c - 1)
        if c >= 2 and c + _NSLOT - 2 < n_chunks:
            wait_wb(c - 2)
            wb_drained.add(c - 2)
            start_d(c + _NSLOT - 2)
    wait_g(n_chunks - 1)
    start_wb(n_chunks - 1)
    for c in range(n_chunks):
        if c not in wb_drained:
            wait_wb(c)


def _sc_gather(dp, pp, ds_t, lo):
    mesh = plsc.VectorSubcoreMesh(core_axis_name="c", subcore_axis_name="s")
    per_w = SEG // 32
    return pl.kernel(
        functools.partial(_sc_gather_body, lo=lo),
        out_type=jax.ShapeDtypeStruct((SEG, HIDDEN), jnp.float32),
        mesh=mesh,
        scratch_types=(
            [pltpu.VMEM((2 * per_w,), jnp.int32)]
            + [pltpu.VMEM((CHUNK, HIDDEN), jnp.float32)] * _NSLOT
            + [pltpu.SemaphoreType.DMA] * (2 * _NSLOT)
        ),
        name=f"sc_gather_seg{lo}",
    )(dp, pp, ds_t)


def _head_body(xs_ref, b1_ref, w2_ref, b2_ref, o_ref):
    h = jnp.tanh(xs_ref[...] + b1_ref[...])
    logits = lax.dot_general(
        w2_ref[...], h,
        dimension_numbers=(((0,), (1,)), ((), ())),
        preferred_element_type=jnp.float32)
    logits += b2_ref[...]
    m = jnp.max(logits, axis=0, keepdims=True)
    lse = m + jnp.log(jnp.sum(jnp.exp(logits - m), axis=0, keepdims=True))
    o_ref[...] = logits - lse


def _tc_head(xs, b1, w2, b2):
    blk = 16384
    return pl.pallas_call(
        _head_body,
        grid=(SEG // blk,),
        in_specs=[
            pl.BlockSpec((blk, HIDDEN), lambda i: (i, 0)),
            pl.BlockSpec((1, HIDDEN), lambda i: (0, 0)),
            pl.BlockSpec((HIDDEN, 2), lambda i: (0, 0)),
            pl.BlockSpec((2, 1), lambda i: (0, 0)),
        ],
        out_specs=pl.BlockSpec((2, blk), lambda i: (0, i)),
        out_shape=jax.ShapeDtypeStruct((2, SEG), jnp.float32),
        compiler_params=pltpu.CompilerParams(
            dimension_semantics=("arbitrary",),
        ),
    )(xs, b1, w2, b2)


def kernel(graph, dataset_index, iftrain, d, p, W1, b1, W2, b2):
    ds_t = dataset_index.astype(jnp.int32).T
    dp, pp = _tc_embed(d.T, p.T, W1[:FEAT], W1[FEAT:])
    b1r = b1.reshape(1, HIDDEN)
    b2r = b2.reshape(2, 1)
    outs = []
    for s in range(N_SEG):
        xs = _sc_gather(dp, pp, ds_t, s * SEG)
        outs.append(_tc_head(xs, b1r, W2, b2r))
    return jnp.concatenate(outs, axis=1).T
